# Initial kernel scaffold; baseline (speedup 1.0000x reference)
import jax
import jax.numpy as jnp
from jax.experimental import pallas as pl


def kernel(pillar_features, voxel_coords, batch_size, mlp_w1, mlp_b1, mlp_w2, mlp_b2, sa0_wq, sa0_wk, sa0_wv, sa0_wo, sa1_wq, sa1_wk, sa1_wv, sa1_wo, sa2_wq, sa2_wk, sa2_wv, sa2_wo):
    raise NotImplementedError("write your pallas kernel here")


# speedup vs baseline: 2.8776x; 2.8776x over previous
import jax
import jax.numpy as jnp
import numpy as np
from jax import lax
from jax.experimental import pallas as pl

N = 12000
C = 64
K = 1024
S = 16
NX, NY, NZ = 432, 496, 1
VX, VY, VZ = 0.16, 0.16, 4.0
X_OFF, Y_OFF, Z_OFF = 0.08 + 0.0, 0.08 - 39.68, 2.0 - 3.0
R2 = 1.2 ** 2
NPAD = 12032
NROWS = NPAD // 128
NPAD_ATT = 12288
BLK_ATT = 1536
KBLK = 128
BIGI = (1 << 30)


def _fps_kernel(xs_ref, ys_ref, zs_ref, out_ref):
    lane = lax.broadcasted_iota(jnp.int32, (1, 128), 1)
    flat = (lax.broadcasted_iota(jnp.int32, (NROWS, 128), 0) * 128
            + lax.broadcasted_iota(jnp.int32, (NROWS, 128), 1))
    dists0 = jnp.where(flat < N, jnp.float32(1e10), jnp.float32(-1.0))

    def body(i, carry):
        dists, far = carry
        ro = i // 128
        co = i % 128
        row = out_ref[pl.ds(ro, 1), :]
        out_ref[pl.ds(ro, 1), :] = jnp.where(lane == co, far, row)
        r = far // 128
        c = far % 128
        selc = (lane == c).astype(jnp.float32)
        xf = jnp.sum(xs_ref[pl.ds(r, 1), :] * selc)
        yf = jnp.sum(ys_ref[pl.ds(r, 1), :] * selc)
        zf = jnp.sum(zs_ref[pl.ds(r, 1), :] * selc)
        dx = xs_ref[...] - xf
        dy = ys_ref[...] - yf
        dz = zs_ref[...] - zf
        d = (dx * dx + dy * dy) + dz * dz
        dists = jnp.minimum(dists, d)
        m = jnp.max(dists)
        far2 = jnp.min(jnp.where(dists >= m, flat, BIGI))
        return dists, far2

    lax.fori_loop(0, K, body, (dists0, jnp.int32(0)))


def _knn_kernel(kpc_ref, px_ref, py_ref, pz_ref, nbr_ref, nbd_ref):
    kb = kpc_ref[...]
    dx = kb[:, 0:1] - px_ref[0:1, :]
    dy = kb[:, 1:2] - py_ref[0:1, :]
    dz = kb[:, 2:3] - pz_ref[0:1, :]
    d2 = (dx * dx + dy * dy) + dz * dz
    li = lax.broadcasted_iota(jnp.int32, (KBLK, NPAD), 1)
    d2v = jnp.where(li < N, d2, jnp.float32(3.0e38))
    for s in range(S):
        m = jnp.min(d2v, axis=1, keepdims=True)
        im = jnp.min(jnp.where(d2v <= m, li, BIGI), axis=1, keepdims=True)
        nbr_ref[:, s:s + 1] = im
        nbd_ref[:, s:s + 1] = m
        d2v = jnp.where(li == im, jnp.float32(3.0e38), d2v)


def _mlp_kernel(g_ref, nbd_ref, w1_ref, b1_ref, w2_ref, b2_ref, out_ref):
    w1 = w1_ref[...]
    w2 = w2_ref[...]
    b1 = b1_ref[0:1, 0:C]
    b2 = b2_ref[0:1, 0:C]
    acc = None
    for s in range(S):
        gs = g_ref[s]
        h = jnp.maximum(
            jnp.dot(gs, w1, preferred_element_type=jnp.float32) + b1, 0.0)
        h2 = jnp.maximum(
            jnp.dot(h, w2, preferred_element_type=jnp.float32) + b2, 0.0)
        msk = (nbd_ref[:, s:s + 1] <= R2).astype(jnp.float32)
        v = h2 * msk
        acc = v if acc is None else jnp.maximum(acc, v)
    out_ref[...] = acc


def _softmax_rows(x):
    m = jnp.max(x, axis=-1, keepdims=True)
    e = jnp.exp(x - m)
    return e / jnp.sum(e, axis=-1, keepdims=True)


def _nt(a, b):
    return lax.dot_general(a, b, (((1,), (1,)), ((), ())),
                           preferred_element_type=jnp.float32)


def _sa_self_kernel(x_ref, wq_ref, wk_ref, wv_ref, wo_ref, out_ref):
    x = x_ref[...]
    q = jnp.dot(x, wq_ref[...], preferred_element_type=jnp.float32)
    k = jnp.dot(x, wk_ref[...], preferred_element_type=jnp.float32)
    v = jnp.dot(x, wv_ref[...], preferred_element_type=jnp.float32)
    attn = _softmax_rows(_nt(q, k) / 8.0)
    ctx = jnp.dot(attn, v, preferred_element_type=jnp.float32)
    out_ref[...] = x + jnp.dot(ctx, wo_ref[...],
                               preferred_element_type=jnp.float32)


def _cross_kernel(p_ref, y_ref,
                  w1q_ref, w1k_ref, w1v_ref, w1o_ref,
                  w2q_ref, w2k_ref, w2v_ref, w2o_ref, out_ref):
    y = y_ref[...]
    x = p_ref[...]
    for wq, wk, wv, wo in ((w1q_ref, w1k_ref, w1v_ref, w1o_ref),
                           (w2q_ref, w2k_ref, w2v_ref, w2o_ref)):
        q = jnp.dot(x, wq[...], preferred_element_type=jnp.float32)
        k = jnp.dot(y, wk[...], preferred_element_type=jnp.float32)
        v = jnp.dot(y, wv[...], preferred_element_type=jnp.float32)
        attn = _softmax_rows(_nt(q, k) / 8.0)
        ctx = jnp.dot(attn, v, preferred_element_type=jnp.float32)
        x = x + jnp.dot(ctx, wo[...], preferred_element_type=jnp.float32)
    out_ref[...] = x


def kernel(pillar_features, voxel_coords, batch_size, mlp_w1, mlp_b1, mlp_w2,
           mlp_b2, sa0_wq, sa0_wk, sa0_wv, sa0_wo, sa1_wq, sa1_wk, sa1_wv,
           sa1_wo, sa2_wq, sa2_wk, sa2_wv, sa2_wo):
    coords = voxel_coords
    pcx = coords[:, 3].astype(jnp.float32) * VX + X_OFF
    pcy = coords[:, 2].astype(jnp.float32) * VY + Y_OFF
    pcz = coords[:, 1].astype(jnp.float32) * VZ + Z_OFF
    pc = jnp.stack([pcx, pcy, pcz], axis=1)

    pad1 = NPAD - N
    xs = jnp.pad(pcx, (0, pad1)).reshape(NROWS, 128)
    ys = jnp.pad(pcy, (0, pad1)).reshape(NROWS, 128)
    zs = jnp.pad(pcz, (0, pad1)).reshape(NROWS, 128)

    kp_idx = pl.pallas_call(
        _fps_kernel,
        out_shape=jax.ShapeDtypeStruct((K // 128, 128), jnp.int32),
    )(xs, ys, zs).reshape(K)

    kp = jnp.take(pc, kp_idx, axis=0)
    kpc = jnp.pad(kp, ((0, 0), (0, 128 - 3)))
    rowx = jnp.broadcast_to(jnp.pad(pcx, (0, pad1))[None, :], (8, NPAD))
    rowy = jnp.broadcast_to(jnp.pad(pcy, (0, pad1))[None, :], (8, NPAD))
    rowz = jnp.broadcast_to(jnp.pad(pcz, (0, pad1))[None, :], (8, NPAD))

    nbr_full, nbd_full = pl.pallas_call(
        _knn_kernel,
        grid=(K // KBLK,),
        in_specs=[
            pl.BlockSpec((KBLK, 128), lambda i: (i, 0)),
            pl.BlockSpec((8, NPAD), lambda i: (0, 0)),
            pl.BlockSpec((8, NPAD), lambda i: (0, 0)),
            pl.BlockSpec((8, NPAD), lambda i: (0, 0)),
        ],
        out_specs=[
            pl.BlockSpec((KBLK, 128), lambda i: (i, 0)),
            pl.BlockSpec((KBLK, 128), lambda i: (i, 0)),
        ],
        out_shape=[
            jax.ShapeDtypeStruct((K, 128), jnp.int32),
            jax.ShapeDtypeStruct((K, 128), jnp.float32),
        ],
    )(kpc, rowx, rowy, rowz)
    nbr = nbr_full[:, :S]

    grouped_xyz = jnp.take(pc, nbr.reshape(-1), axis=0).reshape(K, S, 3) \
        - kp[:, None, :]
    grouped_feat = jnp.take(pillar_features, nbr.reshape(-1),
                            axis=0).reshape(K, S, C)
    g = jnp.concatenate([grouped_xyz, grouped_feat], axis=-1)
    g = jnp.pad(g, ((0, 0), (0, 0), (0, 128 - (C + 3))))
    g = jnp.transpose(g, (1, 0, 2))

    w1p = jnp.pad(mlp_w1, ((0, 128 - (C + 3)), (0, 0)))
    b1p = jnp.zeros((8, 128), jnp.float32).at[0, :C].set(mlp_b1)
    b2p = jnp.zeros((8, 128), jnp.float32).at[0, :C].set(mlp_b2)

    local = pl.pallas_call(
        _mlp_kernel,
        grid=(K // KBLK,),
        in_specs=[
            pl.BlockSpec((S, KBLK, 128), lambda i: (0, i, 0)),
            pl.BlockSpec((KBLK, 128), lambda i: (i, 0)),
            pl.BlockSpec((128, C), lambda i: (0, 0)),
            pl.BlockSpec((8, 128), lambda i: (0, 0)),
            pl.BlockSpec((C, C), lambda i: (0, 0)),
            pl.BlockSpec((8, 128), lambda i: (0, 0)),
        ],
        out_specs=pl.BlockSpec((KBLK, C), lambda i: (i, 0)),
        out_shape=jax.ShapeDtypeStruct((K, C), jnp.float32),
    )(g, nbd_full, w1p, b1p, mlp_w2, b2p)

    local_sa = pl.pallas_call(
        _sa_self_kernel,
        out_shape=jax.ShapeDtypeStruct((K, C), jnp.float32),
    )(local, sa0_wq.T, sa0_wk.T, sa0_wv.T, sa0_wo.T)

    pf_pad = jnp.concatenate(
        [pillar_features, jnp.zeros((NPAD_ATT - N, C), jnp.float32)], axis=0)

    ctx_t = pl.pallas_call(
        _cross_kernel,
        grid=(NPAD_ATT // BLK_ATT,),
        in_specs=[pl.BlockSpec((BLK_ATT, C), lambda i: (i, 0)),
                  pl.BlockSpec((K, C), lambda i: (0, 0))] +
                 [pl.BlockSpec((C, C), lambda i: (0, 0))] * 8,
        out_specs=pl.BlockSpec((BLK_ATT, C), lambda i: (i, 0)),
        out_shape=jax.ShapeDtypeStruct((NPAD_ATT, C), jnp.float32),
    )(pf_pad, local_sa, sa1_wq.T, sa1_wk.T, sa1_wv.T, sa1_wo.T,
      sa2_wq.T, sa2_wk.T, sa2_wv.T, sa2_wo.T)
    context_t = ctx_t[:N]

    idx_flat = coords[:, 1] + coords[:, 2] * NX + coords[:, 3]
    spatial_t = jnp.zeros((NZ * NX * NY, C), jnp.float32).at[idx_flat].set(
        context_t)
    spatial = spatial_t.T.reshape(1, C, NY, NX)
    p0 = jax.image.resize(spatial, (1, C, NY // 2, NX // 2), 'bilinear')
    p1 = jax.image.resize(spatial, (1, C, NY // 4, NX // 4), 'bilinear')
    p2 = jax.image.resize(spatial, (1, C, NY // 8, NX // 8), 'bilinear')
    return (p0, p1, p2)


# speedup vs baseline: 3.0445x; 1.0580x over previous
import jax
import jax.numpy as jnp
import numpy as np
from jax import lax
from jax.experimental import pallas as pl
from jax.experimental.pallas import tpu as pltpu

N = 12000
C = 64
K = 1024
S = 16
NX, NY, NZ = 432, 496, 1
VX, VY, VZ = 0.16, 0.16, 4.0
X_OFF, Y_OFF, Z_OFF = 0.08 + 0.0, 0.08 - 39.68, 2.0 - 3.0
R2 = 1.2 ** 2
NPAD = 12032
NROWS = NPAD // 128
NPAD_ATT = 12288
BLK_ATT = 1536
KBLK = 128
BIGI = (1 << 30)


def _fps_kernel(xs_ref, ys_ref, zs_ref, xsm_ref, ysm_ref, zsm_ref, out_ref):
    flat = (lax.broadcasted_iota(jnp.int32, (NROWS, 128), 0) * 128
            + lax.broadcasted_iota(jnp.int32, (NROWS, 128), 1))
    dists0 = jnp.where(flat < N, jnp.float32(1e10), jnp.float32(-1.0))

    def body(i, carry):
        dists, far = carry
        out_ref[i] = far
        xf = xsm_ref[far]
        yf = ysm_ref[far]
        zf = zsm_ref[far]
        dx = xs_ref[...] - xf
        dy = ys_ref[...] - yf
        dz = zs_ref[...] - zf
        d = (dx * dx + dy * dy) + dz * dz
        dists = jnp.minimum(dists, d)
        m = jnp.max(dists, keepdims=True)
        far2 = jnp.min(jnp.where(dists >= m, flat, BIGI))
        return dists, far2

    lax.fori_loop(0, K, body, (dists0, jnp.int32(0)))


def _knn_kernel(kpc_ref, px_ref, py_ref, pz_ref, nbr_ref, nbd_ref):
    kb = kpc_ref[...]
    dx = kb[:, 0:1] - px_ref[0:1, :]
    dy = kb[:, 1:2] - py_ref[0:1, :]
    dz = kb[:, 2:3] - pz_ref[0:1, :]
    d2 = (dx * dx + dy * dy) + dz * dz
    li = lax.broadcasted_iota(jnp.int32, (KBLK, NPAD), 1)
    d2v = jnp.where(li < N, d2, jnp.float32(3.0e38))
    for s in range(S):
        m = jnp.min(d2v, axis=1, keepdims=True)
        im = jnp.min(jnp.where(d2v <= m, li, BIGI), axis=1, keepdims=True)
        nbr_ref[:, s:s + 1] = im
        nbd_ref[:, s:s + 1] = m
        d2v = jnp.where(li == im, jnp.float32(3.0e38), d2v)


def _mlp_kernel(g_ref, nbd_ref, w1_ref, b1_ref, w2_ref, b2_ref, out_ref):
    w1 = w1_ref[...]
    w2 = w2_ref[...]
    b1 = b1_ref[0:1, 0:C]
    b2 = b2_ref[0:1, 0:C]
    acc = None
    for s in range(S):
        gs = g_ref[s]
        h = jnp.maximum(
            jnp.dot(gs, w1, preferred_element_type=jnp.float32) + b1, 0.0)
        h2 = jnp.maximum(
            jnp.dot(h, w2, preferred_element_type=jnp.float32) + b2, 0.0)
        msk = (nbd_ref[:, s:s + 1] <= R2).astype(jnp.float32)
        v = h2 * msk
        acc = v if acc is None else jnp.maximum(acc, v)
    out_ref[...] = acc


def _softmax_rows(x):
    m = jnp.max(x, axis=-1, keepdims=True)
    e = jnp.exp(x - m)
    return e / jnp.sum(e, axis=-1, keepdims=True)


def _nt(a, b):
    return lax.dot_general(a, b, (((1,), (1,)), ((), ())),
                           preferred_element_type=jnp.float32)


def _sa_self_kernel(x_ref, wq_ref, wk_ref, wv_ref, wo_ref, out_ref):
    x = x_ref[...]
    q = jnp.dot(x, wq_ref[...], preferred_element_type=jnp.float32)
    k = jnp.dot(x, wk_ref[...], preferred_element_type=jnp.float32)
    v = jnp.dot(x, wv_ref[...], preferred_element_type=jnp.float32)
    attn = _softmax_rows(_nt(q, k) / 8.0)
    ctx = jnp.dot(attn, v, preferred_element_type=jnp.float32)
    out_ref[...] = x + jnp.dot(ctx, wo_ref[...],
                               preferred_element_type=jnp.float32)


def _cross_kernel(p_ref, y_ref,
                  w1q_ref, w1k_ref, w1v_ref, w1o_ref,
                  w2q_ref, w2k_ref, w2v_ref, w2o_ref, out_ref):
    y = y_ref[...]
    x = p_ref[...]
    for wq, wk, wv, wo in ((w1q_ref, w1k_ref, w1v_ref, w1o_ref),
                           (w2q_ref, w2k_ref, w2v_ref, w2o_ref)):
        q = jnp.dot(x, wq[...], preferred_element_type=jnp.float32)
        k = jnp.dot(y, wk[...], preferred_element_type=jnp.float32)
        v = jnp.dot(y, wv[...], preferred_element_type=jnp.float32)
        attn = _softmax_rows(_nt(q, k) / 8.0)
        ctx = jnp.dot(attn, v, preferred_element_type=jnp.float32)
        x = x + jnp.dot(ctx, wo[...], preferred_element_type=jnp.float32)
    out_ref[...] = x


def kernel(pillar_features, voxel_coords, batch_size, mlp_w1, mlp_b1, mlp_w2,
           mlp_b2, sa0_wq, sa0_wk, sa0_wv, sa0_wo, sa1_wq, sa1_wk, sa1_wv,
           sa1_wo, sa2_wq, sa2_wk, sa2_wv, sa2_wo):
    coords = voxel_coords
    pcx = coords[:, 3].astype(jnp.float32) * VX + X_OFF
    pcy = coords[:, 2].astype(jnp.float32) * VY + Y_OFF
    pcz = coords[:, 1].astype(jnp.float32) * VZ + Z_OFF
    pc = jnp.stack([pcx, pcy, pcz], axis=1)

    pad1 = NPAD - N
    xs = jnp.pad(pcx, (0, pad1)).reshape(NROWS, 128)
    ys = jnp.pad(pcy, (0, pad1)).reshape(NROWS, 128)
    zs = jnp.pad(pcz, (0, pad1)).reshape(NROWS, 128)

    kp_idx = pl.pallas_call(
        _fps_kernel,
        in_specs=[pl.BlockSpec(memory_space=pltpu.VMEM)] * 3 +
                 [pl.BlockSpec(memory_space=pltpu.SMEM)] * 3,
        out_specs=pl.BlockSpec(memory_space=pltpu.SMEM),
        out_shape=jax.ShapeDtypeStruct((K,), jnp.int32),
    )(xs, ys, zs, jnp.pad(pcx, (0, pad1)), jnp.pad(pcy, (0, pad1)),
      jnp.pad(pcz, (0, pad1)))

    kp = jnp.take(pc, kp_idx, axis=0)
    kpc = jnp.pad(kp, ((0, 0), (0, 128 - 3)))
    rowx = jnp.broadcast_to(jnp.pad(pcx, (0, pad1))[None, :], (8, NPAD))
    rowy = jnp.broadcast_to(jnp.pad(pcy, (0, pad1))[None, :], (8, NPAD))
    rowz = jnp.broadcast_to(jnp.pad(pcz, (0, pad1))[None, :], (8, NPAD))

    nbr_full, nbd_full = pl.pallas_call(
        _knn_kernel,
        grid=(K // KBLK,),
        in_specs=[
            pl.BlockSpec((KBLK, 128), lambda i: (i, 0)),
            pl.BlockSpec((8, NPAD), lambda i: (0, 0)),
            pl.BlockSpec((8, NPAD), lambda i: (0, 0)),
            pl.BlockSpec((8, NPAD), lambda i: (0, 0)),
        ],
        out_specs=[
            pl.BlockSpec((KBLK, 128), lambda i: (i, 0)),
            pl.BlockSpec((KBLK, 128), lambda i: (i, 0)),
        ],
        out_shape=[
            jax.ShapeDtypeStruct((K, 128), jnp.int32),
            jax.ShapeDtypeStruct((K, 128), jnp.float32),
        ],
    )(kpc, rowx, rowy, rowz)
    nbr = nbr_full[:, :S]

    grouped_xyz = jnp.take(pc, nbr.reshape(-1), axis=0).reshape(K, S, 3) \
        - kp[:, None, :]
    grouped_feat = jnp.take(pillar_features, nbr.reshape(-1),
                            axis=0).reshape(K, S, C)
    g = jnp.concatenate([grouped_xyz, grouped_feat], axis=-1)
    g = jnp.pad(g, ((0, 0), (0, 0), (0, 128 - (C + 3))))
    g = jnp.transpose(g, (1, 0, 2))

    w1p = jnp.pad(mlp_w1, ((0, 128 - (C + 3)), (0, 0)))
    b1p = jnp.zeros((8, 128), jnp.float32).at[0, :C].set(mlp_b1)
    b2p = jnp.zeros((8, 128), jnp.float32).at[0, :C].set(mlp_b2)

    local = pl.pallas_call(
        _mlp_kernel,
        grid=(K // KBLK,),
        in_specs=[
            pl.BlockSpec((S, KBLK, 128), lambda i: (0, i, 0)),
            pl.BlockSpec((KBLK, 128), lambda i: (i, 0)),
            pl.BlockSpec((128, C), lambda i: (0, 0)),
            pl.BlockSpec((8, 128), lambda i: (0, 0)),
            pl.BlockSpec((C, C), lambda i: (0, 0)),
            pl.BlockSpec((8, 128), lambda i: (0, 0)),
        ],
        out_specs=pl.BlockSpec((KBLK, C), lambda i: (i, 0)),
        out_shape=jax.ShapeDtypeStruct((K, C), jnp.float32),
    )(g, nbd_full, w1p, b1p, mlp_w2, b2p)

    local_sa = pl.pallas_call(
        _sa_self_kernel,
        out_shape=jax.ShapeDtypeStruct((K, C), jnp.float32),
    )(local, sa0_wq.T, sa0_wk.T, sa0_wv.T, sa0_wo.T)

    pf_pad = jnp.concatenate(
        [pillar_features, jnp.zeros((NPAD_ATT - N, C), jnp.float32)], axis=0)

    ctx_t = pl.pallas_call(
        _cross_kernel,
        grid=(NPAD_ATT // BLK_ATT,),
        in_specs=[pl.BlockSpec((BLK_ATT, C), lambda i: (i, 0)),
                  pl.BlockSpec((K, C), lambda i: (0, 0))] +
                 [pl.BlockSpec((C, C), lambda i: (0, 0))] * 8,
        out_specs=pl.BlockSpec((BLK_ATT, C), lambda i: (i, 0)),
        out_shape=jax.ShapeDtypeStruct((NPAD_ATT, C), jnp.float32),
    )(pf_pad, local_sa, sa1_wq.T, sa1_wk.T, sa1_wv.T, sa1_wo.T,
      sa2_wq.T, sa2_wk.T, sa2_wv.T, sa2_wo.T)
    context_t = ctx_t[:N]

    idx_flat = coords[:, 1] + coords[:, 2] * NX + coords[:, 3]
    spatial_t = jnp.zeros((NZ * NX * NY, C), jnp.float32).at[idx_flat].set(
        context_t)
    spatial = spatial_t.T.reshape(1, C, NY, NX)
    p0 = jax.image.resize(spatial, (1, C, NY // 2, NX // 2), 'bilinear')
    p1 = jax.image.resize(spatial, (1, C, NY // 4, NX // 4), 'bilinear')
    p2 = jax.image.resize(spatial, (1, C, NY // 8, NX // 8), 'bilinear')
    return (p0, p1, p2)
